# baseline (speedup 1.0000x reference)
import functools

import jax
import jax.numpy as jnp
from jax import lax
from jax.experimental import pallas as pl
from jax.experimental.pallas import tpu as pltpu
from jax.experimental.pallas import tpu_sc as plsc

K = 8
NSCALE = 12
NBLOCK = 3
NGRAPH = NSCALE // 2
TAU = 10.0
FEAT_INTERIM = 32

NP_PAD = 10240
KNN_TILE = 512
PAD_COORD = 1e30


def _knn_kernel(q_ref, xyt_ref, out_ref):
    t = pl.program_id(1)
    q = q_ref[0]
    qx = q[:, 0:1]
    qy = q[:, 1:2]
    X = xyt_ref[0, 0:1, :]
    Y = xyt_ref[0, 1:2, :]
    dx = qx - X
    dy = qy - Y
    d = dx * dx + dy * dy
    col = lax.broadcasted_iota(jnp.int32, (KNN_TILE, NP_PAD), 1)
    rowid = t * KNN_TILE + lax.broadcasted_iota(jnp.int32, (KNN_TILE, NP_PAD), 0)
    inf = jnp.float32(jnp.inf)
    d = jnp.where(col == rowid, inf, d)
    for k in range(K):
        idx = jnp.argmin(d, axis=1).astype(jnp.int32)
        out_ref[0, k, :] = idx
        d = jnp.where(col == idx[:, None], inf, d)


def _knn_pallas(d1_xy):
    Bv, Nv, _ = d1_xy.shape
    q = jnp.pad(d1_xy, ((0, 0), (0, NP_PAD - Nv), (0, 0)))
    xyt = jnp.pad(d1_xy.transpose(0, 2, 1), ((0, 0), (0, 0), (0, NP_PAD - Nv)),
                  constant_values=PAD_COORD)
    grid = (Bv, NP_PAD // KNN_TILE)
    nbr = pl.pallas_call(
        _knn_kernel,
        grid=grid,
        in_specs=[
            pl.BlockSpec((1, KNN_TILE, 2), lambda b, t: (b, t, 0)),
            pl.BlockSpec((1, 2, NP_PAD), lambda b, t: (b, 0, 0)),
        ],
        out_specs=pl.BlockSpec((1, K, KNN_TILE), lambda b, t: (b, 0, t)),
        out_shape=jax.ShapeDtypeStruct((Bv, K, NP_PAD), jnp.int32),
    )(q, xyt)
    return nbr[:, :, :Nv]


def _leaky(x):
    return jnp.where(x >= 0, x, 0.2 * x)


_NW = 32


@functools.partial(jax.jit, static_argnames=("chunk",))
def _sc_gather(table, idx, chunk=1000):
    L = idx.shape[0]
    W = table.shape[1]
    lw = L // _NW
    nchunk = lw // chunk
    mesh = plsc.VectorSubcoreMesh(core_axis_name="c", subcore_axis_name="s")

    @functools.partial(
        pl.kernel,
        out_type=jax.ShapeDtypeStruct((L, W), jnp.float32),
        mesh=mesh,
        scratch_types=[
            pltpu.VMEM((2, chunk), jnp.int32),
            pltpu.VMEM((2, chunk, W), jnp.float32),
            pltpu.SemaphoreType.DMA,
            pltpu.SemaphoreType.DMA,
            pltpu.SemaphoreType.DMA,
            pltpu.SemaphoreType.DMA,
        ],
        compiler_params=pltpu.CompilerParams(use_tc_tiling_on_sc=False),
    )
    def k(table_hbm, idx_hbm, out_hbm, idx_v, rows_v, g0, g1, w0, w1):
        gsem = (g0, g1)
        wsem = (w0, w1)
        wid = lax.axis_index("s") * 2 + lax.axis_index("c")
        base = wid * lw
        gh = [None, None]
        wh = [None, None]
        for c in range(nchunk):
            p = c % 2
            if wh[p] is not None:
                wh[p].wait()
            b0 = base + c * chunk
            pltpu.sync_copy(idx_hbm.at[pl.ds(b0, chunk)], idx_v.at[p])
            gh[p] = pltpu.async_copy(table_hbm.at[idx_v.at[p]], rows_v.at[p],
                                     gsem[p])
            q = 1 - p
            if gh[q] is not None:
                gh[q].wait()
                b1 = base + (c - 1) * chunk
                wh[q] = pltpu.async_copy(rows_v.at[q],
                                         out_hbm.at[pl.ds(b1, chunk)], wsem[q])
                gh[q] = None
        p = (nchunk - 1) % 2
        gh[p].wait()
        b1 = base + (nchunk - 1) * chunk
        wh[p] = pltpu.async_copy(rows_v.at[p], out_hbm.at[pl.ds(b1, chunk)],
                                 wsem[p])
        for p in range(2):
            if wh[p] is not None:
                wh[p].wait()

    return k(table, idx)


def _pad_w(w):
    return ((w + 15) // 16) * 16


_ROW_T = 2000


def _table_kernel(x_ref, w_ref, b_ref, asrc_ref, adst_ref, out_ref, *, fout, wg):
    x = x_ref[...]
    h = jnp.dot(x, w_ref[...], preferred_element_type=jnp.float32) + b_ref[...]
    es = jnp.dot(h, asrc_ref[...], preferred_element_type=jnp.float32)
    ed = jnp.dot(h, adst_ref[...], preferred_element_type=jnp.float32)
    pad = jnp.zeros((x.shape[0], wg - fout - 2), jnp.float32)
    out_ref[...] = jnp.concatenate([h, es, ed, pad], axis=1)


def _tc_table(x, p):
    R, fin = x.shape
    fout = p['W'].shape[1]
    wg = _pad_w(fout + 2)
    grid = (R // _ROW_T,)
    return pl.pallas_call(
        functools.partial(_table_kernel, fout=fout, wg=wg),
        grid=grid,
        in_specs=[
            pl.BlockSpec((_ROW_T, fin), lambda r: (r, 0)),
            pl.BlockSpec((fin, fout), lambda r: (0, 0)),
            pl.BlockSpec((1, fout), lambda r: (0, 0)),
            pl.BlockSpec((fout, 1), lambda r: (0, 0)),
            pl.BlockSpec((fout, 1), lambda r: (0, 0)),
        ],
        out_specs=pl.BlockSpec((_ROW_T, wg), lambda r: (r, 0)),
        out_shape=jax.ShapeDtypeStruct((R, wg), jnp.float32),
    )(x, p['W'], p['b'].reshape(1, fout), p['a_src'].reshape(fout, 1),
      p['a_dst'].reshape(fout, 1))


def _attend(g, ha, fout):
    es = g[:, :, fout:fout + 1]
    ed = ha[:, fout + 1:fout + 2]
    e = _leaky(ed[None] + es)
    m = jnp.max(e, axis=0, keepdims=True)
    ex = jnp.exp(e - m)
    s = jnp.sum(ex, axis=0, keepdims=True)
    alpha = ex / s
    return jnp.sum(alpha * g[:, :, :fout], axis=0)


def _mk_table(x, w_ref, b_ref, asrc_ref, adst_ref, wg):
    fout = w_ref.shape[1]
    h = jnp.dot(x, w_ref[...], preferred_element_type=jnp.float32) + b_ref[...]
    es = jnp.dot(h, asrc_ref[...], preferred_element_type=jnp.float32)
    ed = jnp.dot(h, adst_ref[...], preferred_element_type=jnp.float32)
    pad = jnp.zeros((x.shape[0], wg - fout - 2), jnp.float32)
    return jnp.concatenate([h, es, ed, pad], axis=1)


def _pspecs(p):
    fin, fout = p['W'].shape
    return [
        pl.BlockSpec((fin, fout), lambda b, t: (0, 0)),
        pl.BlockSpec((1, fout), lambda b, t: (0, 0)),
        pl.BlockSpec((fout, 1), lambda b, t: (0, 0)),
        pl.BlockSpec((fout, 1), lambda b, t: (0, 0)),
    ]


def _pargs(p):
    fin, fout = p['W'].shape
    return (p['W'], p['b'].reshape(1, fout), p['a_src'].reshape(fout, 1),
            p['a_dst'].reshape(fout, 1))


def _gspec(wg):
    return pl.BlockSpec((1, K, _ROW_T, wg), lambda b, t: (b, 0, t, 0))


def _haspec(wg):
    return pl.BlockSpec((1, _ROW_T, wg), lambda b, t: (b, t, 0))


def _tspec(wg, nt):
    return pl.BlockSpec((_ROW_T, wg), lambda b, t: (b * nt + t, 0))


def _att_tables_kernel(g_ref, ha_ref, *refs, fout, wgs):
    nparam = 4 * len(wgs)
    prefs, orefs = refs[:nparam], refs[nparam:]
    y = _leaky(_attend(g_ref[0], ha_ref[0], fout))
    for j, wg in enumerate(wgs):
        orefs[j][...] = _mk_table(y, *prefs[4 * j:4 * j + 4], wg)


def _att_tables(G, haug, fout, plist):
    Bv, _, Nv, wg = G.shape
    nt = Nv // _ROW_T
    R = Bv * Nv
    wgs = tuple(_pad_w(p['W'].shape[1] + 2) for p in plist)
    specs = [_gspec(wg), _haspec(wg)]
    args = [G, haug.reshape(Bv, Nv, wg)]
    for p in plist:
        specs += _pspecs(p)
        args += list(_pargs(p))
    outs = pl.pallas_call(
        functools.partial(_att_tables_kernel, fout=fout, wgs=wgs),
        grid=(Bv, nt),
        in_specs=specs,
        out_specs=[_tspec(w, nt) for w in wgs],
        out_shape=[jax.ShapeDtypeStruct((R, w), jnp.float32) for w in wgs],
    )(*args)
    return outs


def _att_y_kernel(g_ref, ha_ref, out_ref, *, fout):
    out_ref[0] = _attend(g_ref[0], ha_ref[0], fout)


def _att_y(G, haug, fout):
    Bv, _, Nv, wg = G.shape
    return pl.pallas_call(
        functools.partial(_att_y_kernel, fout=fout),
        grid=(Bv, Nv // _ROW_T),
        in_specs=[_gspec(wg), _haspec(wg)],
        out_specs=_haspec(fout),
        out_shape=jax.ShapeDtypeStruct((Bv, Nv, fout), jnp.float32),
    )(G, haug.reshape(Bv, Nv, wg))


def _masked_softmax_pick(z, evenmask, lane):
    neg = jnp.float32(-jnp.inf)
    m = jnp.max(jnp.where(evenmask, z, neg), axis=1, keepdims=True)
    ex = jnp.where(evenmask, jnp.exp(z - m), 0.0)
    s = jnp.sum(ex, axis=1, keepdims=True)
    y = ex / s
    my = jnp.max(y, axis=1, keepdims=True)
    l = jnp.min(jnp.where(y == my, lane, jnp.int32(99)), axis=1, keepdims=True)
    return l


def _gumbel_kernel(g_ref, ha_ref, d1_ref, gn_ref, *refs, fout, wgs):
    nparam = 4 * len(wgs)
    prefs, orefs = refs[:nparam], refs[nparam + 1:]
    xs_ref = refs[nparam]
    w1 = _attend(g_ref[0], ha_ref[0], fout)
    z = (w1 + gn_ref[0]) / TAU
    lane = lax.broadcasted_iota(jnp.int32, (_ROW_T, NSCALE), 1)
    even = (lane % 2) == 0
    l1 = _masked_softmax_pick(z, even, lane)
    l2 = _masked_softmax_pick(z, ~even, lane)
    d1 = d1_ref[0]
    lane14 = lax.broadcasted_iota(jnp.int32, (_ROW_T, NSCALE + 2), 1)
    x1 = jnp.sum(jnp.where(lane14 == l1 + 2, d1, 0.0), axis=1, keepdims=True)
    x2 = jnp.sum(jnp.where(lane14 == l2 + 2, d1, 0.0), axis=1, keepdims=True)
    xs_ref[0] = jnp.concatenate([x1, x2], axis=1)
    if wgs:
        xsel = jnp.where(even, x1, x2)
        xf = jnp.abs(d1[:, 2:NSCALE + 2] - xsel)
        orefs[0][...] = _mk_table(xf, *prefs[0:4], wgs[0])


def _att_gumbel(G, haug, d1c, gn, plist):
    Bv, _, Nv, wg = G.shape
    nt = Nv // _ROW_T
    R = Bv * Nv
    wgs = tuple(_pad_w(p['W'].shape[1] + 2) for p in plist)
    specs = [_gspec(wg), _haspec(wg), _haspec(NSCALE + 2), _haspec(NSCALE)]
    args = [G, haug.reshape(Bv, Nv, wg), d1c, gn]
    for p in plist:
        specs += _pspecs(p)
        args += list(_pargs(p))
    out_specs = [_haspec(2)] + [_tspec(w, nt) for w in wgs]
    out_shape = [jax.ShapeDtypeStruct((Bv, Nv, 2), jnp.float32)] + [
        jax.ShapeDtypeStruct((R, w), jnp.float32) for w in wgs]
    return pl.pallas_call(
        functools.partial(_gumbel_kernel, fout=NSCALE, wgs=wgs),
        grid=(Bv, nt),
        in_specs=specs,
        out_specs=out_specs,
        out_shape=out_shape,
    )(*args)


def _combine_kernel(g_ref, ha_ref, xw_ref, xs_ref, d1_ref, w_ref, b_ref,
                    asrc_ref, adst_ref, d1n_ref, t_ref, *, fout, wg2):
    dw = _attend(g_ref[0], ha_ref[0], fout)
    xw = xw_ref[0]
    m = jnp.maximum(xw, dw)
    ea = jnp.exp(xw - m)
    eb = jnp.exp(dw - m)
    s = ea + eb
    wa = ea / s
    wb = eb / s
    lane = lax.broadcasted_iota(jnp.int32, (_ROW_T, NSCALE), 1)
    even = (lane % 2) == 0
    xsv = xs_ref[0]
    x1 = xsv[:, 0:1]
    x2 = xsv[:, 1:2]
    xsel = jnp.where(even, x1, x2)
    d1 = d1_ref[0]
    d12 = wa * xsel + wb * d1[:, 2:NSCALE + 2]
    d1n_ref[0] = jnp.concatenate([d1[:, 0:2], d12], axis=1)
    t_ref[...] = _mk_table(d12, w_ref, b_ref, asrc_ref, adst_ref, wg2)


def _att_combine(G, haug, x1_ws, xsi, d1c, pnext):
    Bv, _, Nv, wg = G.shape
    nt = Nv // _ROW_T
    R = Bv * Nv
    wg2 = _pad_w(pnext['W'].shape[1] + 2)
    specs = [_gspec(wg), _haspec(wg), _haspec(NSCALE), _haspec(2),
             _haspec(NSCALE + 2)] + _pspecs(pnext)
    args = [G, haug.reshape(Bv, Nv, wg), x1_ws, xsi, d1c] + list(_pargs(pnext))
    return pl.pallas_call(
        functools.partial(_combine_kernel, fout=NSCALE, wg2=wg2),
        grid=(Bv, nt),
        in_specs=specs,
        out_specs=[_haspec(NSCALE + 2), _tspec(wg2, nt)],
        out_shape=[jax.ShapeDtypeStruct((Bv, Nv, NSCALE + 2), jnp.float32),
                   jax.ShapeDtypeStruct((R, wg2), jnp.float32)],
    )(*args)


def _interleave(a, b):
    return jnp.stack([a, b], axis=3).reshape(a.shape[0], a.shape[1], -1)


@functools.lru_cache(maxsize=4)
def _gnoise(Bv, Nv):
    import numpy as np
    _ctx = jax.ensure_compile_time_eval()
    _ctx.__enter__()
    gkey = jax.random.key(42)
    out = []
    for i in range(NBLOCK):
        u1 = jax.random.uniform(jax.random.fold_in(gkey, 2 * i),
                                (Bv, Nv, NGRAPH), minval=1e-6, maxval=1.0 - 1e-6)
        u2 = jax.random.uniform(jax.random.fold_in(gkey, 2 * i + 1),
                                (Bv, Nv, NGRAPH), minval=1e-6, maxval=1.0 - 1e-6)
        g1 = -jnp.log(-jnp.log(u1))
        g2 = -jnp.log(-jnp.log(u2))
        out.append(np.asarray(_interleave(g1, g2)))
    _ctx.__exit__(None, None, None)
    return tuple(out)


def kernel(d1, params):
    Bv, Nv, _ = d1.shape
    R = Bv * Nv
    nbr = _knn_pallas(d1[:, :, 0:2])
    boff = (jnp.arange(Bv, dtype=jnp.int32) * Nv)[:, None, None]
    nbr = (nbr + boff).reshape(-1)

    def gather(table):
        wg = table.shape[1]
        return _sc_gather(table, nbr).reshape(Bv, K, Nv, wg)

    gns = [jnp.asarray(a) for a in _gnoise(Bv, Nv)]

    d1c = d1
    xs = []
    t_a0 = _tc_table(d1[:, :, 2:].reshape(R, NSCALE), params['attn0']['conv0'])
    for i in range(NBLOCK):
        pa = params['attn%d' % (2 * i)]
        pb = params['attn%d' % (2 * i + 1)]
        last = i == NBLOCK - 1
        (t_a1,) = _att_tables(gather(t_a0), t_a0, FEAT_INTERIM, [pa['conv1']])
        nxt = [pb['conv0']] + ([] if last else [params['exp%d' % (3 * i + 2)]['conv0']])
        outs = _att_tables(gather(t_a1), t_a1, NSCALE, nxt)
        t_b0 = outs[0]
        t_e2_0 = None if last else outs[1]
        (t_b1,) = _att_tables(gather(t_b0), t_b0, FEAT_INTERIM, [pb['conv1']])
        gouts = _att_gumbel(gather(t_b1), t_b1, d1c, gns[i],
                            [] if last else [params['exp%d' % (3 * i)]['conv0']])
        xs.append(gouts[0])
        if last:
            break
        t_x0 = gouts[1]
        pe0 = params['exp%d' % (3 * i)]
        pe1 = params['exp%d' % (3 * i + 1)]
        pe2 = params['exp%d' % (3 * i + 2)]
        (t_x1,) = _att_tables(gather(t_x0), t_x0, FEAT_INTERIM, [pe0['conv1']])
        (t_e1_0,) = _att_tables(gather(t_x1), t_x1, NSCALE, [pe1['conv0']])
        (t_e1_1,) = _att_tables(gather(t_e1_0), t_e1_0, FEAT_INTERIM, [pe1['conv1']])
        x1_ws = _att_y(gather(t_e1_1), t_e1_1, NSCALE)
        (t_e2_1,) = _att_tables(gather(t_e2_0), t_e2_0, FEAT_INTERIM, [pe2['conv1']])
        d1c, t_a0 = _att_combine(gather(t_e2_1), t_e2_1, x1_ws, xs[-1], d1c,
                                 params['attn%d' % (2 * i + 2)]['conv0'])
    return jnp.stack(xs, axis=0)


# speedup vs baseline: 1.0759x; 1.0759x over previous
import functools

import jax
import jax.numpy as jnp
from jax import lax
from jax.experimental import pallas as pl
from jax.experimental.pallas import tpu as pltpu
from jax.experimental.pallas import tpu_sc as plsc

K = 8
NSCALE = 12
NBLOCK = 3
NGRAPH = NSCALE // 2
TAU = 10.0
FEAT_INTERIM = 32

NP_PAD = 10240
KNN_TILE = 128
PAD_COORD = 1e30


def _knn_kernel(q_ref, xyt_ref, out_ref):
    t = pl.program_id(1)
    q = q_ref[0]
    qx = q[:, 0:1]
    qy = q[:, 1:2]
    X = xyt_ref[0, 0:1, :]
    Y = xyt_ref[0, 1:2, :]
    dx = qx - X
    dy = qy - Y
    d = dx * dx + dy * dy
    col = lax.broadcasted_iota(jnp.int32, (KNN_TILE, NP_PAD), 1)
    rowid = t * KNN_TILE + lax.broadcasted_iota(jnp.int32, (KNN_TILE, NP_PAD), 0)
    inf = jnp.float32(jnp.inf)
    d = jnp.where(col == rowid, inf, d)
    for k in range(K):
        idx = jnp.argmin(d, axis=1).astype(jnp.int32)
        out_ref[0, k, :] = idx
        d = jnp.where(col == idx[:, None], inf, d)


def _knn_pallas(d1_xy):
    Bv, Nv, _ = d1_xy.shape
    q = jnp.pad(d1_xy, ((0, 0), (0, NP_PAD - Nv), (0, 0)))
    xyt = jnp.pad(d1_xy.transpose(0, 2, 1), ((0, 0), (0, 0), (0, NP_PAD - Nv)),
                  constant_values=PAD_COORD)
    grid = (Bv, NP_PAD // KNN_TILE)
    nbr = pl.pallas_call(
        _knn_kernel,
        grid=grid,
        in_specs=[
            pl.BlockSpec((1, KNN_TILE, 2), lambda b, t: (b, t, 0)),
            pl.BlockSpec((1, 2, NP_PAD), lambda b, t: (b, 0, 0)),
        ],
        out_specs=pl.BlockSpec((1, K, KNN_TILE), lambda b, t: (b, 0, t)),
        out_shape=jax.ShapeDtypeStruct((Bv, K, NP_PAD), jnp.int32),
    )(q, xyt)
    return nbr[:, :, :Nv]


def _leaky(x):
    return jnp.where(x >= 0, x, 0.2 * x)


_NW = 32


@functools.partial(jax.jit, static_argnames=("chunk",))
def _sc_gather(table, idx, chunk=1000):
    L = idx.shape[0]
    W = table.shape[1]
    lw = L // _NW
    nchunk = lw // chunk
    mesh = plsc.VectorSubcoreMesh(core_axis_name="c", subcore_axis_name="s")

    @functools.partial(
        pl.kernel,
        out_type=jax.ShapeDtypeStruct((L, W), jnp.float32),
        mesh=mesh,
        scratch_types=[
            pltpu.VMEM((2, chunk), jnp.int32),
            pltpu.VMEM((2, chunk, W), jnp.float32),
            pltpu.SemaphoreType.DMA,
            pltpu.SemaphoreType.DMA,
            pltpu.SemaphoreType.DMA,
            pltpu.SemaphoreType.DMA,
        ],
        compiler_params=pltpu.CompilerParams(use_tc_tiling_on_sc=False),
    )
    def k(table_hbm, idx_hbm, out_hbm, idx_v, rows_v, g0, g1, w0, w1):
        gsem = (g0, g1)
        wsem = (w0, w1)
        wid = lax.axis_index("s") * 2 + lax.axis_index("c")
        base = wid * lw
        gh = [None, None]
        wh = [None, None]
        for c in range(nchunk):
            p = c % 2
            if wh[p] is not None:
                wh[p].wait()
            b0 = base + c * chunk
            pltpu.sync_copy(idx_hbm.at[pl.ds(b0, chunk)], idx_v.at[p])
            gh[p] = pltpu.async_copy(table_hbm.at[idx_v.at[p]], rows_v.at[p],
                                     gsem[p])
            q = 1 - p
            if gh[q] is not None:
                gh[q].wait()
                b1 = base + (c - 1) * chunk
                wh[q] = pltpu.async_copy(rows_v.at[q],
                                         out_hbm.at[pl.ds(b1, chunk)], wsem[q])
                gh[q] = None
        p = (nchunk - 1) % 2
        gh[p].wait()
        b1 = base + (nchunk - 1) * chunk
        wh[p] = pltpu.async_copy(rows_v.at[p], out_hbm.at[pl.ds(b1, chunk)],
                                 wsem[p])
        for p in range(2):
            if wh[p] is not None:
                wh[p].wait()

    return k(table, idx)


def _pad_w(w):
    return ((w + 15) // 16) * 16


_ROW_T = 2000


def _table_kernel(x_ref, w_ref, b_ref, asrc_ref, adst_ref, out_ref, *, fout, wg):
    x = x_ref[...]
    h = jnp.dot(x, w_ref[...], preferred_element_type=jnp.float32) + b_ref[...]
    es = jnp.dot(h, asrc_ref[...], preferred_element_type=jnp.float32)
    ed = jnp.dot(h, adst_ref[...], preferred_element_type=jnp.float32)
    pad = jnp.zeros((x.shape[0], wg - fout - 2), jnp.float32)
    out_ref[...] = jnp.concatenate([h, es, ed, pad], axis=1)


def _tc_table(x, p):
    R, fin = x.shape
    fout = p['W'].shape[1]
    wg = _pad_w(fout + 2)
    grid = (R // _ROW_T,)
    return pl.pallas_call(
        functools.partial(_table_kernel, fout=fout, wg=wg),
        grid=grid,
        in_specs=[
            pl.BlockSpec((_ROW_T, fin), lambda r: (r, 0)),
            pl.BlockSpec((fin, fout), lambda r: (0, 0)),
            pl.BlockSpec((1, fout), lambda r: (0, 0)),
            pl.BlockSpec((fout, 1), lambda r: (0, 0)),
            pl.BlockSpec((fout, 1), lambda r: (0, 0)),
        ],
        out_specs=pl.BlockSpec((_ROW_T, wg), lambda r: (r, 0)),
        out_shape=jax.ShapeDtypeStruct((R, wg), jnp.float32),
    )(x, p['W'], p['b'].reshape(1, fout), p['a_src'].reshape(fout, 1),
      p['a_dst'].reshape(fout, 1))


def _attend(g, ha, fout):
    es = g[:, :, fout:fout + 1]
    ed = ha[:, fout + 1:fout + 2]
    e = _leaky(ed[None] + es)
    m = jnp.max(e, axis=0, keepdims=True)
    ex = jnp.exp(e - m)
    s = jnp.sum(ex, axis=0, keepdims=True)
    alpha = ex / s
    return jnp.sum(alpha * g[:, :, :fout], axis=0)


def _mk_table(x, w_ref, b_ref, asrc_ref, adst_ref, wg):
    fout = w_ref.shape[1]
    h = jnp.dot(x, w_ref[...], preferred_element_type=jnp.float32) + b_ref[...]
    es = jnp.dot(h, asrc_ref[...], preferred_element_type=jnp.float32)
    ed = jnp.dot(h, adst_ref[...], preferred_element_type=jnp.float32)
    pad = jnp.zeros((x.shape[0], wg - fout - 2), jnp.float32)
    return jnp.concatenate([h, es, ed, pad], axis=1)


def _pspecs(p):
    fin, fout = p['W'].shape
    return [
        pl.BlockSpec((fin, fout), lambda b, t: (0, 0)),
        pl.BlockSpec((1, fout), lambda b, t: (0, 0)),
        pl.BlockSpec((fout, 1), lambda b, t: (0, 0)),
        pl.BlockSpec((fout, 1), lambda b, t: (0, 0)),
    ]


def _pargs(p):
    fin, fout = p['W'].shape
    return (p['W'], p['b'].reshape(1, fout), p['a_src'].reshape(fout, 1),
            p['a_dst'].reshape(fout, 1))


def _gspec(wg):
    return pl.BlockSpec((1, K, _ROW_T, wg), lambda b, t: (b, 0, t, 0))


def _haspec(wg):
    return pl.BlockSpec((1, _ROW_T, wg), lambda b, t: (b, t, 0))


def _tspec(wg, nt):
    return pl.BlockSpec((_ROW_T, wg), lambda b, t: (b * nt + t, 0))


def _att_tables_kernel(g_ref, ha_ref, *refs, fout, wgs):
    nparam = 4 * len(wgs)
    prefs, orefs = refs[:nparam], refs[nparam:]
    y = _leaky(_attend(g_ref[0], ha_ref[0], fout))
    for j, wg in enumerate(wgs):
        orefs[j][...] = _mk_table(y, *prefs[4 * j:4 * j + 4], wg)


def _att_tables(G, haug, fout, plist):
    Bv, _, Nv, wg = G.shape
    nt = Nv // _ROW_T
    R = Bv * Nv
    wgs = tuple(_pad_w(p['W'].shape[1] + 2) for p in plist)
    specs = [_gspec(wg), _haspec(wg)]
    args = [G, haug.reshape(Bv, Nv, wg)]
    for p in plist:
        specs += _pspecs(p)
        args += list(_pargs(p))
    outs = pl.pallas_call(
        functools.partial(_att_tables_kernel, fout=fout, wgs=wgs),
        grid=(Bv, nt),
        in_specs=specs,
        out_specs=[_tspec(w, nt) for w in wgs],
        out_shape=[jax.ShapeDtypeStruct((R, w), jnp.float32) for w in wgs],
    )(*args)
    return outs


def _att_y_kernel(g_ref, ha_ref, out_ref, *, fout):
    out_ref[0] = _attend(g_ref[0], ha_ref[0], fout)


def _att_y(G, haug, fout):
    Bv, _, Nv, wg = G.shape
    return pl.pallas_call(
        functools.partial(_att_y_kernel, fout=fout),
        grid=(Bv, Nv // _ROW_T),
        in_specs=[_gspec(wg), _haspec(wg)],
        out_specs=_haspec(fout),
        out_shape=jax.ShapeDtypeStruct((Bv, Nv, fout), jnp.float32),
    )(G, haug.reshape(Bv, Nv, wg))


def _masked_softmax_pick(z, evenmask, lane):
    neg = jnp.float32(-jnp.inf)
    m = jnp.max(jnp.where(evenmask, z, neg), axis=1, keepdims=True)
    ex = jnp.where(evenmask, jnp.exp(z - m), 0.0)
    s = jnp.sum(ex, axis=1, keepdims=True)
    y = ex / s
    my = jnp.max(y, axis=1, keepdims=True)
    l = jnp.min(jnp.where(y == my, lane, jnp.int32(99)), axis=1, keepdims=True)
    return l


def _gumbel_kernel(g_ref, ha_ref, d1_ref, gn_ref, *refs, fout, wgs):
    nparam = 4 * len(wgs)
    prefs, orefs = refs[:nparam], refs[nparam + 1:]
    xs_ref = refs[nparam]
    w1 = _attend(g_ref[0], ha_ref[0], fout)
    z = (w1 + gn_ref[0]) / TAU
    lane = lax.broadcasted_iota(jnp.int32, (_ROW_T, NSCALE), 1)
    even = (lane % 2) == 0
    l1 = _masked_softmax_pick(z, even, lane)
    l2 = _masked_softmax_pick(z, ~even, lane)
    d1 = d1_ref[0]
    lane14 = lax.broadcasted_iota(jnp.int32, (_ROW_T, NSCALE + 2), 1)
    x1 = jnp.sum(jnp.where(lane14 == l1 + 2, d1, 0.0), axis=1, keepdims=True)
    x2 = jnp.sum(jnp.where(lane14 == l2 + 2, d1, 0.0), axis=1, keepdims=True)
    xs_ref[0] = jnp.concatenate([x1, x2], axis=1)
    if wgs:
        xsel = jnp.where(even, x1, x2)
        xf = jnp.abs(d1[:, 2:NSCALE + 2] - xsel)
        orefs[0][...] = _mk_table(xf, *prefs[0:4], wgs[0])


def _att_gumbel(G, haug, d1c, gn, plist):
    Bv, _, Nv, wg = G.shape
    nt = Nv // _ROW_T
    R = Bv * Nv
    wgs = tuple(_pad_w(p['W'].shape[1] + 2) for p in plist)
    specs = [_gspec(wg), _haspec(wg), _haspec(NSCALE + 2), _haspec(NSCALE)]
    args = [G, haug.reshape(Bv, Nv, wg), d1c, gn]
    for p in plist:
        specs += _pspecs(p)
        args += list(_pargs(p))
    out_specs = [_haspec(2)] + [_tspec(w, nt) for w in wgs]
    out_shape = [jax.ShapeDtypeStruct((Bv, Nv, 2), jnp.float32)] + [
        jax.ShapeDtypeStruct((R, w), jnp.float32) for w in wgs]
    return pl.pallas_call(
        functools.partial(_gumbel_kernel, fout=NSCALE, wgs=wgs),
        grid=(Bv, nt),
        in_specs=specs,
        out_specs=out_specs,
        out_shape=out_shape,
    )(*args)


def _combine_kernel(g_ref, ha_ref, xw_ref, xs_ref, d1_ref, w_ref, b_ref,
                    asrc_ref, adst_ref, d1n_ref, t_ref, *, fout, wg2):
    dw = _attend(g_ref[0], ha_ref[0], fout)
    xw = xw_ref[0]
    m = jnp.maximum(xw, dw)
    ea = jnp.exp(xw - m)
    eb = jnp.exp(dw - m)
    s = ea + eb
    wa = ea / s
    wb = eb / s
    lane = lax.broadcasted_iota(jnp.int32, (_ROW_T, NSCALE), 1)
    even = (lane % 2) == 0
    xsv = xs_ref[0]
    x1 = xsv[:, 0:1]
    x2 = xsv[:, 1:2]
    xsel = jnp.where(even, x1, x2)
    d1 = d1_ref[0]
    d12 = wa * xsel + wb * d1[:, 2:NSCALE + 2]
    d1n_ref[0] = jnp.concatenate([d1[:, 0:2], d12], axis=1)
    t_ref[...] = _mk_table(d12, w_ref, b_ref, asrc_ref, adst_ref, wg2)


def _att_combine(G, haug, x1_ws, xsi, d1c, pnext):
    Bv, _, Nv, wg = G.shape
    nt = Nv // _ROW_T
    R = Bv * Nv
    wg2 = _pad_w(pnext['W'].shape[1] + 2)
    specs = [_gspec(wg), _haspec(wg), _haspec(NSCALE), _haspec(2),
             _haspec(NSCALE + 2)] + _pspecs(pnext)
    args = [G, haug.reshape(Bv, Nv, wg), x1_ws, xsi, d1c] + list(_pargs(pnext))
    return pl.pallas_call(
        functools.partial(_combine_kernel, fout=NSCALE, wg2=wg2),
        grid=(Bv, nt),
        in_specs=specs,
        out_specs=[_haspec(NSCALE + 2), _tspec(wg2, nt)],
        out_shape=[jax.ShapeDtypeStruct((Bv, Nv, NSCALE + 2), jnp.float32),
                   jax.ShapeDtypeStruct((R, wg2), jnp.float32)],
    )(*args)


def _interleave(a, b):
    return jnp.stack([a, b], axis=3).reshape(a.shape[0], a.shape[1], -1)


@functools.lru_cache(maxsize=4)
def _gnoise(Bv, Nv):
    import numpy as np
    _ctx = jax.ensure_compile_time_eval()
    _ctx.__enter__()
    gkey = jax.random.key(42)
    out = []
    for i in range(NBLOCK):
        u1 = jax.random.uniform(jax.random.fold_in(gkey, 2 * i),
                                (Bv, Nv, NGRAPH), minval=1e-6, maxval=1.0 - 1e-6)
        u2 = jax.random.uniform(jax.random.fold_in(gkey, 2 * i + 1),
                                (Bv, Nv, NGRAPH), minval=1e-6, maxval=1.0 - 1e-6)
        g1 = -jnp.log(-jnp.log(u1))
        g2 = -jnp.log(-jnp.log(u2))
        out.append(np.asarray(_interleave(g1, g2)))
    _ctx.__exit__(None, None, None)
    return tuple(out)


def kernel(d1, params):
    Bv, Nv, _ = d1.shape
    R = Bv * Nv
    nbr = _knn_pallas(d1[:, :, 0:2])
    boff = (jnp.arange(Bv, dtype=jnp.int32) * Nv)[:, None, None]
    nbr = (nbr + boff).reshape(-1)

    def gather(table):
        wg = table.shape[1]
        return _sc_gather(table, nbr).reshape(Bv, K, Nv, wg)

    gns = [jnp.asarray(a) for a in _gnoise(Bv, Nv)]

    d1c = d1
    xs = []
    t_a0 = _tc_table(d1[:, :, 2:].reshape(R, NSCALE), params['attn0']['conv0'])
    for i in range(NBLOCK):
        pa = params['attn%d' % (2 * i)]
        pb = params['attn%d' % (2 * i + 1)]
        last = i == NBLOCK - 1
        (t_a1,) = _att_tables(gather(t_a0), t_a0, FEAT_INTERIM, [pa['conv1']])
        nxt = [pb['conv0']] + ([] if last else [params['exp%d' % (3 * i + 2)]['conv0']])
        outs = _att_tables(gather(t_a1), t_a1, NSCALE, nxt)
        t_b0 = outs[0]
        t_e2_0 = None if last else outs[1]
        (t_b1,) = _att_tables(gather(t_b0), t_b0, FEAT_INTERIM, [pb['conv1']])
        gouts = _att_gumbel(gather(t_b1), t_b1, d1c, gns[i],
                            [] if last else [params['exp%d' % (3 * i)]['conv0']])
        xs.append(gouts[0])
        if last:
            break
        t_x0 = gouts[1]
        pe0 = params['exp%d' % (3 * i)]
        pe1 = params['exp%d' % (3 * i + 1)]
        pe2 = params['exp%d' % (3 * i + 2)]
        (t_x1,) = _att_tables(gather(t_x0), t_x0, FEAT_INTERIM, [pe0['conv1']])
        (t_e1_0,) = _att_tables(gather(t_x1), t_x1, NSCALE, [pe1['conv0']])
        (t_e1_1,) = _att_tables(gather(t_e1_0), t_e1_0, FEAT_INTERIM, [pe1['conv1']])
        x1_ws = _att_y(gather(t_e1_1), t_e1_1, NSCALE)
        (t_e2_1,) = _att_tables(gather(t_e2_0), t_e2_0, FEAT_INTERIM, [pe2['conv1']])
        d1c, t_a0 = _att_combine(gather(t_e2_1), t_e2_1, x1_ws, xs[-1], d1c,
                                 params['attn%d' % (2 * i + 2)]['conv0'])
    return jnp.stack(xs, axis=0)
